# baseline (speedup 1.0000x reference)
import jax
import jax.numpy as jnp
from jax.experimental import pallas as pl
from jax.experimental.pallas import tpu as pltpu

_B, _T, _DIN = 256, 100, 300
_H = 256
_NC = 3956
_NPAD = 4096
_DHID = 512
_BN = 512
_NBLK = _NPAD // _BN

_f32 = jnp.float32
_bf16 = jnp.bfloat16


_UNROLL = 8
_NSTEPS = 13
_AROWS = 320


def _sig(z):
    return 0.5 + 0.5 * jnp.tanh(0.5 * z)


def _lstm_body(x_ref, adj_ref, wihf_ref, whhf_ref, bf_ref, wihb_ref, bb_ref,
               out_ref, adjp_ref, h_scr, c_scr):
    t = pl.program_id(0)

    @pl.when(t == 0)
    def _init():
        h_scr[...] = jnp.zeros_like(h_scr)
        c_scr[...] = jnp.zeros_like(c_scr)

    a = adj_ref[...]
    col = jax.lax.broadcasted_iota(jnp.int32, (_AROWS, _NPAD), 1)

    @pl.when(t < _NSTEPS - 1)
    def _pad_body():
        adjp_ref[...] = jnp.where(col < _NC, a, 0.0).astype(_bf16)

    @pl.when(t == _NSTEPS - 1)
    def _pad_last():
        row = jax.lax.broadcasted_iota(jnp.int32, (_AROWS, _NPAD), 0) + t * _AROWS
        ok = (col < _NC) & (row < _NC)
        adjp_ref[...] = jnp.where(ok, a, 0.0).astype(_bf16)

    h = h_scr[...]
    c = c_scr[...]
    bias = bf_ref[...]
    xt99 = None
    h99 = None
    for k in range(_UNROLL):
        xt = x_ref[:, k, :].astype(_bf16)
        gates = (
            jax.lax.dot(xt, wihf_ref[...], preferred_element_type=_f32)
            + jax.lax.dot(h, whhf_ref[...], preferred_element_type=_f32)
            + bias
        )
        i = _sig(gates[:, 0 * _H:1 * _H])
        f = _sig(gates[:, 1 * _H:2 * _H])
        g = jnp.tanh(gates[:, 2 * _H:3 * _H])
        o = _sig(gates[:, 3 * _H:4 * _H])
        c_new = f * c + i * g
        h_new = o * jnp.tanh(c_new)
        ok = t * _UNROLL + k < _T
        c = jnp.where(ok, c_new, c)
        h = jnp.where(ok, h_new.astype(_bf16), h)
        if k == (_T - 1) % _UNROLL:
            xt99 = xt
            h99 = h_new
    h_scr[...] = h
    c_scr[...] = c

    @pl.when(t == _NSTEPS - 1)
    def _fin():
        out_ref[:, :_H] = h99
        gb = jax.lax.dot(xt99, wihb_ref[...], preferred_element_type=_f32) + bb_ref[...]
        ib = _sig(gb[:, 0 * _H:1 * _H])
        gg = jnp.tanh(gb[:, 2 * _H:3 * _H])
        ob = _sig(gb[:, 3 * _H:4 * _H])
        out_ref[:, _H:] = ob * jnp.tanh(ib * gg)


def _lstm_features(x, adj, wihf, whhf, bias_f, wihb, bias_b):
    return pl.pallas_call(
        _lstm_body,
        grid=(_NSTEPS,),
        in_specs=[
            pl.BlockSpec((_B, _UNROLL, _DIN), lambda t: (0, t, 0)),
            pl.BlockSpec((_AROWS, _NPAD), lambda t: (t, 0)),
            pl.BlockSpec((_DIN, 4 * _H), lambda t: (0, 0)),
            pl.BlockSpec((_H, 4 * _H), lambda t: (0, 0)),
            pl.BlockSpec((1, 4 * _H), lambda t: (0, 0)),
            pl.BlockSpec((_DIN, 4 * _H), lambda t: (0, 0)),
            pl.BlockSpec((1, 4 * _H), lambda t: (0, 0)),
        ],
        out_specs=[
            pl.BlockSpec((_B, 2 * _H), lambda t: (0, 0)),
            pl.BlockSpec((_AROWS, _NPAD), lambda t: (t, 0)),
        ],
        out_shape=[
            jax.ShapeDtypeStruct((_B, 2 * _H), _f32),
            jax.ShapeDtypeStruct((_NPAD, _NPAD), _bf16),
        ],
        scratch_shapes=[
            pltpu.VMEM((_B, _H), _bf16),
            pltpu.VMEM((_B, _H), _f32),
        ],
        compiler_params=pltpu.CompilerParams(
            dimension_semantics=("arbitrary",),
        ),
    )(x, adj, wihf, whhf, bias_f, wihb, bias_b)


def _make_gat_body(din, elu, final):
    nchunk = 8
    rows = _NPAD // nchunk

    def body(*refs):
        if final:
            (xin_ref, adj_ref, w_ref, adstt_ref, feats_ref,
             out_ref, h_scr, fsrc_scr, fdst_scr) = refs
        else:
            (xin_ref, adj_ref, w_ref, adstt_ref,
             out_ref, h_scr, fsrc_scr, fdst_scr) = refs
        i = pl.program_id(0)

        @pl.when(i == 0)
        def _proj():
            for k in range(nchunk):
                sl = pl.ds(k * rows, rows)
                xk = xin_ref[sl, :]
                hk = jax.lax.dot(xk, w_ref[...], preferred_element_type=_f32)
                hkb = hk.astype(_bf16)
                h_scr[sl, 0:_DHID] = hkb[:, 0:_DHID]
                h_scr[sl, _DHID + 128:2 * _DHID + 128] = hkb[:, _DHID:2 * _DHID]
                h_scr[sl, _DHID:_DHID + 128] = jnp.ones((rows, 128), _bf16)
                h_scr[sl, 2 * _DHID + 128:] = jnp.ones((rows, 128), _bf16)
                fsrc_scr[sl, :] = hk[:, 2 * _DHID:]
                fdst_scr[:, sl] = jax.lax.dot_general(
                    adstt_ref[...], xk, (((1,), (1,)), ((), ())),
                    preferred_element_type=_f32)

        adjb = adj_ref[...]
        row0 = i * _BN
        fsrc_blk = fsrc_scr[pl.ds(row0, _BN), :]
        parts = []
        vw = _DHID + 128
        for hd in range(2):
            f1 = fsrc_blk[:, hd:hd + 1]
            f2 = fdst_scr[hd:hd + 1, :]
            f2m = jnp.max(f2, axis=1, keepdims=True)
            mrow = f1 + f2m
            mrow = jnp.maximum(mrow, 0.2 * mrow)
            u = (f1 - mrow) + f2
            d = jnp.maximum(u, 0.2 * u - 0.8 * mrow)
            pb = jnp.exp(d).astype(_bf16) * adjb
            v_ext = h_scr[:, hd * vw:(hd + 1) * vw]
            acc = jax.lax.dot(pb, v_ext, preferred_element_type=_f32)
            s = jnp.maximum(acc[:, _DHID:_DHID + 1], _f32(1e-30))
            parts.append(acc[:, :_DHID] * (1.0 / s))
        att = 0.5 * (parts[0] + parts[1])
        if elu:
            att = jnp.where(att > 0, att, jnp.exp(att) - 1.0)
        if final:
            logits = jax.lax.dot_general(
                feats_ref[...], att.astype(_bf16), (((1,), (1,)), ((), ())),
                preferred_element_type=_f32)
            out_ref[...] = jax.nn.sigmoid(logits)
        else:
            out_ref[...] = att.astype(_bf16)

    return body


def _gat_layer(xin, adj_p, w_ext, adstt, *, elu, final, feats=None):
    din = xin.shape[1]
    in_specs = [
        pl.BlockSpec((_NPAD, din), lambda i: (0, 0)),
        pl.BlockSpec((_BN, _NPAD), lambda i: (i, 0)),
        pl.BlockSpec((din, 2 * _DHID + 128), lambda i: (0, 0)),
        pl.BlockSpec((128, din), lambda i: (0, 0)),
    ]
    args = [xin, adj_p, w_ext, adstt]
    if final:
        in_specs.append(pl.BlockSpec((_B, 2 * _H), lambda i: (0, 0)))
        args.append(feats)
        out_spec = pl.BlockSpec((_B, _BN), lambda i: (0, i))
        out_shape = jax.ShapeDtypeStruct((_B, _NPAD), _f32)
    else:
        out_spec = pl.BlockSpec((_BN, _DHID), lambda i: (i, 0))
        out_shape = jax.ShapeDtypeStruct((_NPAD, _DHID), _bf16)
    return pl.pallas_call(
        _make_gat_body(din, elu, final),
        grid=(_NBLK,),
        in_specs=in_specs,
        out_specs=out_spec,
        out_shape=out_shape,
        scratch_shapes=[
            pltpu.VMEM((_NPAD, 2 * _DHID + 256), _bf16),
            pltpu.VMEM((_NPAD, 128), _f32),
            pltpu.VMEM((128, _NPAD), _f32),
        ],
        compiler_params=pltpu.CompilerParams(
            dimension_semantics=("arbitrary",),
        ),
    )(*args)


def _gat_weights(W, a):
    din = W.shape[1]
    wsrc = jnp.zeros((din, 128), _f32)
    wsrc = wsrc.at[:, 0].set(W[0] @ a[0, :_DHID])
    wsrc = wsrc.at[:, 1].set(W[1] @ a[1, :_DHID])
    w_ext = jnp.concatenate([W[0], W[1], wsrc], axis=1).astype(_bf16)
    adstt = jnp.zeros((128, din), _f32)
    adstt = adstt.at[0, :].set(W[0] @ a[0, _DHID:])
    adstt = adstt.at[1, :].set(W[1] @ a[1, _DHID:])
    return w_ext, adstt.astype(_bf16)


def kernel(x, feat, adj, Wih_f, Whh_f, bih_f, bhh_f, Wih_b, Whh_b, bih_b,
           bhh_b, W1, a1, W2, a2):
    wihf = Wih_f.T.astype(_bf16)
    whhf = Whh_f.T.astype(_bf16)
    bias_f = (bih_f + bhh_f)[None, :]
    wihb = Wih_b.T.astype(_bf16)
    bias_b = (bih_b + bhh_b)[None, :]

    feat_p = jnp.pad(feat, ((0, _NPAD - _NC), (0, 0))).astype(_bf16)
    w1_ext, adstt1 = _gat_weights(W1, a1)
    w2_ext, adstt2 = _gat_weights(W2, a2)

    features, adj_p = _lstm_features(x, adj, wihf, whhf, bias_f, wihb, bias_b)
    att1 = _gat_layer(feat_p, adj_p, w1_ext, adstt1,
                      elu=True, final=False)
    out_p = _gat_layer(att1, adj_p, w2_ext, adstt2,
                       elu=False, final=True, feats=features.astype(_bf16))
    return out_p[:, :_NC]


# speedup vs baseline: 1.0965x; 1.0965x over previous
import jax
import jax.numpy as jnp
from jax.experimental import pallas as pl
from jax.experimental.pallas import tpu as pltpu

_B, _T, _DIN = 256, 100, 300
_H = 256
_NC = 3956
_NPAD = 4096
_DHID = 512
_BN = 512
_NBLK = _NPAD // _BN

_f32 = jnp.float32
_bf16 = jnp.bfloat16


_UNROLL = 10
_DINE = _DIN + 1


_AROWS = 416


def _sig(z):
    return 0.5 + 0.5 * jnp.tanh(0.5 * z)


def _lstm_body(x_ref, adj_ref, wihf_ref, whhf_ref, wihb_ref,
               out_ref, adjp_ref, h_scr, c_scr):
    t = pl.program_id(0)

    @pl.when(t == 0)
    def _init():
        h_scr[...] = jnp.zeros_like(h_scr)
        c_scr[...] = jnp.zeros_like(c_scr)

    a = adj_ref[...]
    col = jax.lax.broadcasted_iota(jnp.int32, (_AROWS, _NPAD), 1)

    @pl.when(t < _T // _UNROLL - 1)
    def _pad_body():
        adjp_ref[...] = jnp.where(col < _NC, a, 0.0).astype(_bf16)

    @pl.when(t == _T // _UNROLL - 1)
    def _pad_last():
        row = jax.lax.broadcasted_iota(jnp.int32, (_AROWS, _NPAD), 0) + t * _AROWS
        ok = (col < _NC) & (row < _NC)
        adjp_ref[...] = jnp.where(ok, a, 0.0).astype(_bf16)

    h = h_scr[...]
    c = c_scr[...]
    xt = None
    h_f32 = None
    for k in range(_UNROLL):
        xt = x_ref[k]
        gates = (
            jax.lax.dot(xt, wihf_ref[...], preferred_element_type=_f32)
            + jax.lax.dot(h, whhf_ref[...], preferred_element_type=_f32)
        )
        i = _sig(gates[:, 0 * _H:1 * _H])
        f = _sig(gates[:, 1 * _H:2 * _H])
        g = jnp.tanh(gates[:, 2 * _H:3 * _H])
        o = _sig(gates[:, 3 * _H:4 * _H])
        c = f * c + i * g
        h_f32 = o * jnp.tanh(c)
        h = h_f32.astype(_bf16)
    h_scr[...] = h
    c_scr[...] = c

    @pl.when(t == _T // _UNROLL - 1)
    def _fin():
        out_ref[:, :_H] = h_f32
        gb = jax.lax.dot(xt, wihb_ref[...], preferred_element_type=_f32)
        ib = _sig(gb[:, 0 * _H:1 * _H])
        gg = jnp.tanh(gb[:, 2 * _H:3 * _H])
        ob = _sig(gb[:, 3 * _H:4 * _H])
        out_ref[:, _H:] = ob * jnp.tanh(ib * gg)


def _lstm_features(x, adj, wihf, whhf, wihb):
    return pl.pallas_call(
        _lstm_body,
        grid=(_T // _UNROLL,),
        in_specs=[
            pl.BlockSpec((_UNROLL, _B, _DINE), lambda t: (t, 0, 0)),
            pl.BlockSpec((_AROWS, _NPAD), lambda t: (t, 0)),
            pl.BlockSpec((_DINE, 4 * _H), lambda t: (0, 0)),
            pl.BlockSpec((_H, 4 * _H), lambda t: (0, 0)),
            pl.BlockSpec((_DINE, 4 * _H), lambda t: (0, 0)),
        ],
        out_specs=[
            pl.BlockSpec((_B, 2 * _H), lambda t: (0, 0)),
            pl.BlockSpec((_AROWS, _NPAD), lambda t: (t, 0)),
        ],
        out_shape=[
            jax.ShapeDtypeStruct((_B, 2 * _H), _f32),
            jax.ShapeDtypeStruct((_NPAD, _NPAD), _bf16),
        ],
        scratch_shapes=[
            pltpu.VMEM((_B, _H), _bf16),
            pltpu.VMEM((_B, _H), _f32),
        ],
        compiler_params=pltpu.CompilerParams(
            dimension_semantics=("arbitrary",),
        ),
    )(x, adj, wihf, whhf, wihb)


def _make_gat_body(din, elu, final):
    nchunk = 8
    rows = _NPAD // nchunk

    def body(*refs):
        if final:
            (xin_ref, adj_ref, w_ref, adstt_ref, feats_ref,
             out_ref, h_scr, fsrc_scr, fdst_scr) = refs
        else:
            (xin_ref, adj_ref, w_ref, adstt_ref,
             out_ref, h_scr, fsrc_scr, fdst_scr) = refs
        i = pl.program_id(0)

        @pl.when(i == 0)
        def _proj():
            for k in range(nchunk):
                sl = pl.ds(k * rows, rows)
                xk = xin_ref[sl, :]
                hk = jax.lax.dot(xk, w_ref[...], preferred_element_type=_f32)
                hkb = hk.astype(_bf16)
                h_scr[sl, 0:_DHID] = hkb[:, 0:_DHID]
                h_scr[sl, _DHID + 128:2 * _DHID + 128] = hkb[:, _DHID:2 * _DHID]
                h_scr[sl, _DHID:_DHID + 128] = jnp.ones((rows, 128), _bf16)
                h_scr[sl, 2 * _DHID + 128:] = jnp.ones((rows, 128), _bf16)
                fsrc_scr[sl, :] = hk[:, 2 * _DHID:]
                fdst_scr[:, sl] = jax.lax.dot_general(
                    adstt_ref[...], xk, (((1,), (1,)), ((), ())),
                    preferred_element_type=_f32)

        adjb = adj_ref[...]
        row0 = i * _BN
        fsrc_blk = fsrc_scr[pl.ds(row0, _BN), :]
        parts = []
        vw = _DHID + 128
        for hd in range(2):
            f1 = fsrc_blk[:, hd:hd + 1]
            f2 = fdst_scr[hd:hd + 1, :]
            f2m = jnp.max(f2, axis=1, keepdims=True)
            mrow = f1 + f2m
            mrow = jnp.maximum(mrow, 0.2 * mrow)
            u = (f1 - mrow) + f2
            d = jnp.maximum(u, 0.2 * u - 0.8 * mrow)
            pb = jnp.exp(d).astype(_bf16) * adjb
            v_ext = h_scr[:, hd * vw:(hd + 1) * vw]
            acc = jax.lax.dot(pb, v_ext, preferred_element_type=_f32)
            s = jnp.maximum(acc[:, _DHID:_DHID + 1], _f32(1e-30))
            parts.append(acc[:, :_DHID] * (1.0 / s))
        att = 0.5 * (parts[0] + parts[1])
        if elu:
            att = jnp.where(att > 0, att, jnp.exp(att) - 1.0)
        if final:
            logits = jax.lax.dot_general(
                feats_ref[...], att.astype(_bf16), (((1,), (1,)), ((), ())),
                preferred_element_type=_f32)
            out_ref[...] = jax.nn.sigmoid(logits)
        else:
            out_ref[...] = att.astype(_bf16)

    return body


def _gat_layer(xin, adj_p, w_ext, adstt, *, elu, final, feats=None):
    din = xin.shape[1]
    in_specs = [
        pl.BlockSpec((_NPAD, din), lambda i: (0, 0)),
        pl.BlockSpec((_BN, _NPAD), lambda i: (i, 0)),
        pl.BlockSpec((din, 2 * _DHID + 128), lambda i: (0, 0)),
        pl.BlockSpec((128, din), lambda i: (0, 0)),
    ]
    args = [xin, adj_p, w_ext, adstt]
    if final:
        in_specs.append(pl.BlockSpec((_B, 2 * _H), lambda i: (0, 0)))
        args.append(feats)
        out_spec = pl.BlockSpec((_B, _BN), lambda i: (0, i))
        out_shape = jax.ShapeDtypeStruct((_B, _NPAD), _f32)
    else:
        out_spec = pl.BlockSpec((_BN, _DHID), lambda i: (i, 0))
        out_shape = jax.ShapeDtypeStruct((_NPAD, _DHID), _bf16)
    return pl.pallas_call(
        _make_gat_body(din, elu, final),
        grid=(_NBLK,),
        in_specs=in_specs,
        out_specs=out_spec,
        out_shape=out_shape,
        scratch_shapes=[
            pltpu.VMEM((_NPAD, 2 * _DHID + 256), _bf16),
            pltpu.VMEM((_NPAD, 128), _f32),
            pltpu.VMEM((128, _NPAD), _f32),
        ],
        compiler_params=pltpu.CompilerParams(
            dimension_semantics=("arbitrary",),
        ),
    )(*args)


def _gat_weights(W, a):
    din = W.shape[1]
    wsrc = jnp.zeros((din, 128), _f32)
    wsrc = wsrc.at[:, 0].set(W[0] @ a[0, :_DHID])
    wsrc = wsrc.at[:, 1].set(W[1] @ a[1, :_DHID])
    w_ext = jnp.concatenate([W[0], W[1], wsrc], axis=1).astype(_bf16)
    adstt = jnp.zeros((128, din), _f32)
    adstt = adstt.at[0, :].set(W[0] @ a[0, _DHID:])
    adstt = adstt.at[1, :].set(W[1] @ a[1, _DHID:])
    return w_ext, adstt.astype(_bf16)


def kernel(x, feat, adj, Wih_f, Whh_f, bih_f, bhh_f, Wih_b, Whh_b, bih_b,
           bhh_b, W1, a1, W2, a2):
    wihf = jnp.concatenate([Wih_f.T, (bih_f + bhh_f)[None, :]], axis=0).astype(_bf16)
    whhf = Whh_f.T.astype(_bf16)
    wihb = jnp.concatenate([Wih_b.T, (bih_b + bhh_b)[None, :]], axis=0).astype(_bf16)
    xb = jnp.concatenate(
        [jnp.swapaxes(x, 0, 1), jnp.ones((_T, _B, 1), _f32)], axis=-1
    ).astype(_bf16)

    feat_p = jnp.pad(feat, ((0, _NPAD - _NC), (0, 0))).astype(_bf16)
    w1_ext, adstt1 = _gat_weights(W1, a1)
    w2_ext, adstt2 = _gat_weights(W2, a2)

    features, adj_p = _lstm_features(xb, adj, wihf, whhf, wihb)
    att1 = _gat_layer(feat_p, adj_p, w1_ext, adstt1,
                      elu=True, final=False)
    out_p = _gat_layer(att1, adj_p, w2_ext, adstt2,
                       elu=False, final=True, feats=features.astype(_bf16))
    return out_p[:, :_NC]
